# baseline (speedup 1.0000x reference)
import functools

import jax
import jax.numpy as jnp
from jax import lax
from jax.experimental import pallas as pl
from jax.experimental.pallas import tpu as pltpu
from jax.experimental.pallas import tpu_sc as plsc

BATCH = 16384
EMBED = 8
VROWS = 1000001
BLK = 128
SPLIT = (VROWS // BLK) * BLK
NBLK = SPLIT // BLK
REST_PAD = BLK
NUM_CORES = 2
NUM_SUBCORES = 16
NUM_WORKERS = NUM_CORES * NUM_SUBCORES
B_PER_W = BATCH // NUM_WORKERS
OBLK_PER_W = B_PER_W // BLK
W_PER_W = B_PER_W * EMBED
BLK_WORDS = BLK * EMBED
CHUNK = 128
LANES = 16
V_PER_BLK = BLK // LANES

_mesh = plsc.VectorSubcoreMesh(core_axis_name="c", subcore_axis_name="s")


@functools.partial(
    pl.kernel,
    mesh=_mesh,
    out_type=jax.ShapeDtypeStruct((BATCH * EMBED,), jnp.float32),
    scratch_types=[
        pltpu.VMEM((B_PER_W,), jnp.int32),
        pltpu.VMEM((W_PER_W,), jnp.int32),
        pltpu.VMEM((W_PER_W,), jnp.float32),
        pltpu.VMEM((REST_PAD * EMBED,), jnp.float32),
        pltpu.SemaphoreType.DMA,
        pltpu.SemaphoreType.DMA,
    ],
    compiler_params=pltpu.CompilerParams(
        use_tc_tiling_on_sc=False, needs_layout_passes=False
    ),
)
def _gather_rows(idx_hbm, big_hbm, rest_hbm, out_hbm, idx_v, widx_v, rows_v,
                 rest_v, sem, rsem):
    wid = lax.axis_index("s") * NUM_CORES + lax.axis_index("c")
    base = wid * B_PER_W
    rest_cp = pltpu.make_async_copy(rest_hbm, rest_v, rsem)
    rest_cp.start()
    pltpu.sync_copy(idx_hbm.at[pl.ds(base, B_PER_W)], idx_v)
    copies = []
    for b in range(OBLK_PER_W):
        for u in range(V_PER_BLK):
            v = b * V_PER_BLK + u
            a = idx_v[pl.ds(v * LANES, LANES)]
            ab = jnp.where(a < SPLIT, a, 0)
            boff = ((ab >> 7) << 10) + (ab & (BLK - 1))
            for j in range(EMBED):
                p = b * BLK_WORDS + j * BLK + u * LANES
                widx_v[pl.ds(p, LANES)] = boff + j * BLK
        for j in range(EMBED):
            p = b * BLK_WORDS + j * BLK
            copies.append(
                pltpu.async_copy(
                    big_hbm.at[widx_v.at[pl.ds(p, CHUNK)]],
                    rows_v.at[pl.ds(p, CHUNK)],
                    sem,
                )
            )
    for cp in copies:
        cp.wait()
    rest_cp.wait()
    for v in range(B_PER_W // LANES):
        a = idx_v[pl.ds(v * LANES, LANES)]
        m = a >= SPLIT

        @pl.when(jnp.any(m))
        def _patch(a=a, m=m, v=v):
            r = (a - SPLIT) * EMBED
            b, u = divmod(v, V_PER_BLK)
            for j in range(EMBED):
                p = b * BLK_WORDS + j * BLK + u * LANES
                val = plsc.load_gather(rest_v, [r + j], mask=m)
                cur = rows_v[pl.ds(p, LANES)]
                rows_v[pl.ds(p, LANES)] = jnp.where(m, val, cur)

    pltpu.sync_copy(rows_v, out_hbm.at[pl.ds(wid * W_PER_W, W_PER_W)])


def kernel(user_id, table):
    big = (
        table[:SPLIT]
        .T.reshape(EMBED, NBLK, BLK)
        .transpose(1, 0, 2)
        .reshape(-1)
    )
    rest = jnp.pad(table[SPLIT:], ((0, REST_PAD - (VROWS - SPLIT)), (0, 0)))
    out = _gather_rows(user_id, big, rest.reshape(-1))
    return (
        out.reshape(BATCH // BLK, EMBED, BLK)
        .transpose(1, 0, 2)
        .reshape(EMBED, BATCH)
        .T
    )
